# baseline (speedup 1.0000x reference)
import functools
import jax
import jax.numpy as jnp
from jax import lax
from jax.experimental import pallas as pl
from jax.experimental.pallas import tpu as pltpu
from jax.experimental.pallas import tpu_sc as plsc

R_SC = 12288
BR = 4096
NW = 32
CH_R = 64


def _tc_body(x_ref, m_ref, o_ref):
    o_ref[...] = x_ref[...] + m_ref[...].astype(jnp.float32)


def _sc_add_by_mask(rows, row_off, C):
    per_w = rows // NW
    n_chunks = per_w // CH_R
    n_outer = n_chunks // 2
    mesh = plsc.VectorSubcoreMesh(core_axis_name="c", subcore_axis_name="s")

    @functools.partial(
        pl.kernel,
        mesh=mesh,
        out_type=jax.ShapeDtypeStruct((rows, C), jnp.float32),
        compiler_params=pltpu.CompilerParams(needs_layout_passes=False),
        scratch_types=[
            pltpu.VMEM((2, CH_R, C), jnp.float32),
            pltpu.VMEM((2, CH_R // 4, C), jnp.uint32),
            pltpu.SemaphoreType.DMA((2,)),
            pltpu.SemaphoreType.DMA((2,)),
            pltpu.SemaphoreType.DMA((2,)),
        ],
    )
    def k(x_hbm, m8_hbm, out_hbm, xb, mb, sin_x, sin_m, sout):
        wid = lax.axis_index("s") * 2 + lax.axis_index("c")
        base = row_off + wid * per_w
        obase = wid * per_w
        m32_hbm = m8_hbm.bitcast(jnp.uint32)

        def start_in(ci, slot):
            r0 = base + ci * CH_R
            pltpu.make_async_copy(
                x_hbm.at[pl.ds(r0, CH_R)], xb.at[slot], sin_x.at[slot]
            ).start()
            pltpu.make_async_copy(
                m32_hbm.at[pl.ds(pl.multiple_of(r0 // 4, 16), CH_R // 4)],
                mb.at[slot], sin_m.at[slot]
            ).start()

        def wait_in(slot):
            pltpu.make_async_copy(
                x_hbm.at[pl.ds(0, CH_R)], xb.at[slot], sin_x.at[slot]
            ).wait()
            pltpu.make_async_copy(
                m32_hbm.at[pl.ds(0, CH_R // 4)], mb.at[slot], sin_m.at[slot]
            ).wait()

        def start_out(ci, slot):
            r0 = obase + ci * CH_R
            pltpu.make_async_copy(
                xb.at[slot], out_hbm.at[pl.ds(r0, CH_R)], sout.at[slot]
            ).start()

        def wait_out(slot):
            pltpu.make_async_copy(
                xb.at[slot], out_hbm.at[pl.ds(0, CH_R)], sout.at[slot]
            ).wait()

        def compute(slot):
            def row_body(r4, _):
                for c16 in range(C // 16):
                    w = mb[slot, r4, pl.ds(c16 * 16, 16)]
                    for q in range(4):
                        mj = ((w >> jnp.uint32(8 * q)) & jnp.uint32(1)).astype(
                            jnp.float32
                        )
                        xb[slot, r4 * 4 + q, pl.ds(c16 * 16, 16)] = (
                            xb[slot, r4 * 4 + q, pl.ds(c16 * 16, 16)] + mj
                        )
                return 0

            lax.fori_loop(0, CH_R // 4, row_body, 0)

        start_in(0, 0)

        def outer(oi, _):
            ca = 2 * oi
            cb = 2 * oi + 1

            @pl.when(oi > 0)
            def _():
                wait_out(1)

            start_in(cb, 1)
            wait_in(0)
            compute(0)
            start_out(ca, 0)
            wait_in(1)
            compute(1)
            start_out(cb, 1)

            @pl.when(oi < n_outer - 1)
            def _():
                wait_out(0)
                start_in(ca + 2, 0)

            return 0

        lax.fori_loop(0, n_outer, outer, 0)
        wait_out(0)
        wait_out(1)

    return k


def kernel(x, mask):
    R, C = x.shape
    R_tc = R - R_SC
    m8 = mask.view(jnp.int8)

    out_sc = _sc_add_by_mask(R_SC, R_tc, C)(x, m8)

    out_tc = pl.pallas_call(
        _tc_body,
        grid=(R_tc // BR,),
        in_specs=[
            pl.BlockSpec((BR, C), lambda i: (i, 0)),
            pl.BlockSpec((BR, C), lambda i: (i, 0)),
        ],
        out_specs=pl.BlockSpec((BR, C), lambda i: (i, 0)),
        out_shape=jax.ShapeDtypeStruct((R, C), x.dtype),
    )(x, m8)

    return lax.dynamic_update_slice(out_tc, out_sc, (R_tc, 0))


# speedup vs baseline: 1.0341x; 1.0341x over previous
import functools
import jax
import jax.numpy as jnp
from jax import lax
from jax.experimental import pallas as pl
from jax.experimental.pallas import tpu as pltpu
from jax.experimental.pallas import tpu_sc as plsc

R_SC = 8192
BR = 4096
NW = 32
CH_R = 64


def _tc_body(x_ref, m_ref, o_ref):
    o_ref[...] = x_ref[...] + m_ref[...].astype(jnp.float32)


def _sc_add_by_mask(rows, row_off, C):
    per_w = rows // NW
    n_chunks = per_w // CH_R
    n_outer = n_chunks // 2
    mesh = plsc.VectorSubcoreMesh(core_axis_name="c", subcore_axis_name="s")

    @functools.partial(
        pl.kernel,
        mesh=mesh,
        out_type=jax.ShapeDtypeStruct((rows, C), jnp.float32),
        compiler_params=pltpu.CompilerParams(needs_layout_passes=False),
        scratch_types=[
            pltpu.VMEM((2, CH_R, C), jnp.float32),
            pltpu.VMEM((2, CH_R // 4, C), jnp.uint32),
            pltpu.SemaphoreType.DMA((2,)),
            pltpu.SemaphoreType.DMA((2,)),
            pltpu.SemaphoreType.DMA((2,)),
        ],
    )
    def k(x_hbm, m8_hbm, out_hbm, xb, mb, sin_x, sin_m, sout):
        wid = lax.axis_index("s") * 2 + lax.axis_index("c")
        base = row_off + wid * per_w
        obase = wid * per_w
        m32_hbm = m8_hbm.bitcast(jnp.uint32)

        def start_in(ci, slot):
            r0 = base + ci * CH_R
            pltpu.make_async_copy(
                x_hbm.at[pl.ds(r0, CH_R)], xb.at[slot], sin_x.at[slot]
            ).start()
            pltpu.make_async_copy(
                m32_hbm.at[pl.ds(pl.multiple_of(r0 // 4, 16), CH_R // 4)],
                mb.at[slot], sin_m.at[slot]
            ).start()

        def wait_in(slot):
            pltpu.make_async_copy(
                x_hbm.at[pl.ds(0, CH_R)], xb.at[slot], sin_x.at[slot]
            ).wait()
            pltpu.make_async_copy(
                m32_hbm.at[pl.ds(0, CH_R // 4)], mb.at[slot], sin_m.at[slot]
            ).wait()

        def start_out(ci, slot):
            r0 = obase + ci * CH_R
            pltpu.make_async_copy(
                xb.at[slot], out_hbm.at[pl.ds(r0, CH_R)], sout.at[slot]
            ).start()

        def wait_out(slot):
            pltpu.make_async_copy(
                xb.at[slot], out_hbm.at[pl.ds(0, CH_R)], sout.at[slot]
            ).wait()

        def compute(slot):
            def row_body(r4, _):
                for c16 in range(C // 16):
                    w = mb[slot, r4, pl.ds(c16 * 16, 16)]
                    for q in range(4):
                        mj = ((w >> jnp.uint32(8 * q)) & jnp.uint32(1)).astype(
                            jnp.float32
                        )
                        xb[slot, r4 * 4 + q, pl.ds(c16 * 16, 16)] = (
                            xb[slot, r4 * 4 + q, pl.ds(c16 * 16, 16)] + mj
                        )
                return 0

            lax.fori_loop(0, CH_R // 4, row_body, 0)

        start_in(0, 0)

        def outer(oi, _):
            ca = 2 * oi
            cb = 2 * oi + 1

            @pl.when(oi > 0)
            def _():
                wait_out(1)

            start_in(cb, 1)
            wait_in(0)
            compute(0)
            start_out(ca, 0)
            wait_in(1)
            compute(1)
            start_out(cb, 1)

            @pl.when(oi < n_outer - 1)
            def _():
                wait_out(0)
                start_in(ca + 2, 0)

            return 0

        lax.fori_loop(0, n_outer, outer, 0)
        wait_out(0)
        wait_out(1)

    return k


def kernel(x, mask):
    R, C = x.shape
    R_tc = R - R_SC
    m8 = mask.view(jnp.int8)

    out_sc = _sc_add_by_mask(R_SC, R_tc, C)(x, m8)

    out_tc = pl.pallas_call(
        _tc_body,
        grid=(R_tc // BR,),
        in_specs=[
            pl.BlockSpec((BR, C), lambda i: (i, 0)),
            pl.BlockSpec((BR, C), lambda i: (i, 0)),
        ],
        out_specs=pl.BlockSpec((BR, C), lambda i: (i, 0)),
        out_shape=jax.ShapeDtypeStruct((R, C), x.dtype),
    )(x, m8)

    return lax.dynamic_update_slice(out_tc, out_sc, (R_tc, 0))


# speedup vs baseline: 1.0788x; 1.0432x over previous
import functools
import jax
import jax.numpy as jnp
from jax import lax
from jax.experimental import pallas as pl
from jax.experimental.pallas import tpu as pltpu
from jax.experimental.pallas import tpu_sc as plsc

R_SC = 4096
BR = 4096
NW = 32
CH_R = 64


def _tc_body(x_ref, m_ref, o_ref):
    o_ref[...] = x_ref[...] + m_ref[...].astype(jnp.float32)


def _sc_add_by_mask(rows, row_off, C):
    per_w = rows // NW
    n_chunks = per_w // CH_R
    n_outer = n_chunks // 2
    mesh = plsc.VectorSubcoreMesh(core_axis_name="c", subcore_axis_name="s")

    @functools.partial(
        pl.kernel,
        mesh=mesh,
        out_type=jax.ShapeDtypeStruct((rows, C), jnp.float32),
        compiler_params=pltpu.CompilerParams(needs_layout_passes=False),
        scratch_types=[
            pltpu.VMEM((2, CH_R, C), jnp.float32),
            pltpu.VMEM((2, CH_R // 4, C), jnp.uint32),
            pltpu.SemaphoreType.DMA((2,)),
            pltpu.SemaphoreType.DMA((2,)),
            pltpu.SemaphoreType.DMA((2,)),
        ],
    )
    def k(x_hbm, m8_hbm, out_hbm, xb, mb, sin_x, sin_m, sout):
        wid = lax.axis_index("s") * 2 + lax.axis_index("c")
        base = row_off + wid * per_w
        obase = wid * per_w
        m32_hbm = m8_hbm.bitcast(jnp.uint32)

        def start_in(ci, slot):
            r0 = base + ci * CH_R
            pltpu.make_async_copy(
                x_hbm.at[pl.ds(r0, CH_R)], xb.at[slot], sin_x.at[slot]
            ).start()
            pltpu.make_async_copy(
                m32_hbm.at[pl.ds(pl.multiple_of(r0 // 4, 16), CH_R // 4)],
                mb.at[slot], sin_m.at[slot]
            ).start()

        def wait_in(slot):
            pltpu.make_async_copy(
                x_hbm.at[pl.ds(0, CH_R)], xb.at[slot], sin_x.at[slot]
            ).wait()
            pltpu.make_async_copy(
                m32_hbm.at[pl.ds(0, CH_R // 4)], mb.at[slot], sin_m.at[slot]
            ).wait()

        def start_out(ci, slot):
            r0 = obase + ci * CH_R
            pltpu.make_async_copy(
                xb.at[slot], out_hbm.at[pl.ds(r0, CH_R)], sout.at[slot]
            ).start()

        def wait_out(slot):
            pltpu.make_async_copy(
                xb.at[slot], out_hbm.at[pl.ds(0, CH_R)], sout.at[slot]
            ).wait()

        def compute(slot):
            def row_body(r4, _):
                for c16 in range(C // 16):
                    w = mb[slot, r4, pl.ds(c16 * 16, 16)]
                    for q in range(4):
                        mj = ((w >> jnp.uint32(8 * q)) & jnp.uint32(1)).astype(
                            jnp.float32
                        )
                        xb[slot, r4 * 4 + q, pl.ds(c16 * 16, 16)] = (
                            xb[slot, r4 * 4 + q, pl.ds(c16 * 16, 16)] + mj
                        )
                return 0

            lax.fori_loop(0, CH_R // 4, row_body, 0)

        start_in(0, 0)

        def outer(oi, _):
            ca = 2 * oi
            cb = 2 * oi + 1

            @pl.when(oi > 0)
            def _():
                wait_out(1)

            start_in(cb, 1)
            wait_in(0)
            compute(0)
            start_out(ca, 0)
            wait_in(1)
            compute(1)
            start_out(cb, 1)

            @pl.when(oi < n_outer - 1)
            def _():
                wait_out(0)
                start_in(ca + 2, 0)

            return 0

        lax.fori_loop(0, n_outer, outer, 0)
        wait_out(0)
        wait_out(1)

    return k


def kernel(x, mask):
    R, C = x.shape
    R_tc = R - R_SC
    m8 = mask.view(jnp.int8)

    out_sc = _sc_add_by_mask(R_SC, R_tc, C)(x, m8)

    out_tc = pl.pallas_call(
        _tc_body,
        grid=(R_tc // BR,),
        in_specs=[
            pl.BlockSpec((BR, C), lambda i: (i, 0)),
            pl.BlockSpec((BR, C), lambda i: (i, 0)),
        ],
        out_specs=pl.BlockSpec((BR, C), lambda i: (i, 0)),
        out_shape=jax.ShapeDtypeStruct((R, C), x.dtype),
    )(x, m8)

    return lax.dynamic_update_slice(out_tc, out_sc, (R_tc, 0))
